# Initial kernel scaffold; baseline (speedup 1.0000x reference)
import jax
import jax.numpy as jnp
from jax.experimental import pallas as pl


def kernel(b, d):
    raise NotImplementedError("write your pallas kernel here")


# speedup vs baseline: 67.8595x; 67.8595x over previous
import jax
import jax.numpy as jnp
from jax import lax
from jax.experimental import pallas as pl

_B = 16
_N = 4096
_K = 32
_T = 511
_TPAD = 512


def _cmpex(a, j, k):
    i = lax.broadcasted_iota(jnp.int32, a.shape, a.ndim - 1)
    bit = (i & j) != 0
    desc = (i & k) == 0
    partner = jnp.where(bit, jnp.roll(a, j, axis=a.ndim - 1),
                        jnp.roll(a, -j, axis=a.ndim - 1))
    take_max = desc ^ bit
    return jnp.where(take_max, jnp.maximum(a, partner), jnp.minimum(a, partner))


def _bitonic_desc64(a):
    k = 2
    while k <= 64:
        j = k // 2
        while j >= 1:
            a = _cmpex(a, j, k)
            j //= 2
        k *= 2
    return a


def _landscape_kernel(b_ref, d_ref, out_ref):
    bv = b_ref[:, :]
    dv = d_ref[:, :]
    lane = lax.broadcasted_iota(jnp.int32, (_B, _N), 1)

    def top32(vals, partner, fill):
        work = vals
        tops, parts, idxs = [], [], []
        for _ in range(_K):
            m = jnp.max(work, axis=1, keepdims=True)
            idx = jnp.min(jnp.where(work == m, lane, _N), axis=1, keepdims=True)
            onehot = lane == idx
            pv = jnp.sum(jnp.where(onehot, partner, 0.0), axis=1, keepdims=True)
            tops.append(m)
            parts.append(pv)
            idxs.append(idx)
            work = jnp.where(onehot, fill, work)
        return (jnp.concatenate(tops, axis=1),
                jnp.concatenate(parts, axis=1),
                jnp.concatenate(idxs, axis=1))

    b_top, d_at_b, ib = top32(bv, dv, jnp.float32(-1.0))
    negd_top, b_at_d, idd = top32(-dv, bv, jnp.float32(-2.0))
    d_bot = -negd_top

    keep = jnp.ones_like(b_at_d)
    for i in range(_K):
        keep = keep * jnp.where(idd == ib[:, i:i + 1], 0.0, 1.0)

    tt = lax.broadcasted_iota(jnp.int32, (_B, _TPAD, _K), 1).astype(
        jnp.float32) * (1.0 / _T)
    cand_a = jnp.maximum(b_top.reshape(_B, 1, _K) - tt,
                         tt - d_at_b.reshape(_B, 1, _K))
    cand_a = jnp.maximum(cand_a, 0.0)
    cand_b = jnp.maximum(b_at_d.reshape(_B, 1, _K) - tt,
                         tt - d_bot.reshape(_B, 1, _K))
    cand_b = jnp.maximum(cand_b, 0.0) * keep.reshape(_B, 1, _K)
    cand = jnp.concatenate([cand_a, cand_b], axis=2)

    cand = _bitonic_desc64(cand)
    out_ref[:, :, :] = cand[:, :, :_K]


def kernel(b, d):
    out = pl.pallas_call(
        _landscape_kernel,
        out_shape=jax.ShapeDtypeStruct((_B, _TPAD, _K), jnp.float32),
    )(b, d)
    return jnp.swapaxes(out[:, :_T, :], 1, 2)
